# baseline (speedup 1.0000x reference)
import functools

import jax
import jax.numpy as jnp
from jax import lax
from jax.experimental import pallas as pl
from jax.experimental.pallas import tpu as pltpu
from jax.experimental.pallas import tpu_sc as plsc

VOCAB = 1000000
EMBED_DIM = 15
EMBED_PAD = 16
BATCH = 16384
FIELDS = 100
N_ROWS = BATCH * FIELDS
PACK = 8
N_PACKED = N_ROWS // PACK

_NC = 2
_NS = 16
_NW = _NC * _NS
_FIRE = 8
_BLK = 128
_CHUNK = _FIRE * _BLK
_PER_W = N_ROWS // _NW
_STEPS = _PER_W // _CHUNK

_PAD_CH = 4000
_PAD_NCH = VOCAB // _PAD_CH
_PAD_ITERS = -(-_PAD_NCH // _NW)
_PAD_UNROLL = 8

_SC_MESH = dict(core_axis_name="c", subcore_axis_name="s")


def _sc_pad(table_flat_cm):
    mesh = plsc.VectorSubcoreMesh(**_SC_MESH)
    in_w = _PAD_CH * EMBED_DIM
    out_w = _PAD_CH * EMBED_PAD

    @functools.partial(
        pl.kernel,
        mesh=mesh,
        out_type=jax.ShapeDtypeStruct((VOCAB * EMBED_PAD,), jnp.float32),
        scratch_types=[
            pltpu.VMEM((in_w,), jnp.float32),
            pltpu.VMEM((out_w,), jnp.float32),
            pltpu.SemaphoreType.DMA,
        ],
        compiler_params=pltpu.CompilerParams(
            use_tc_tiling_on_sc=False, needs_layout_passes=False),
    )
    def pad_kernel(src_hbm, out_hbm, buf_in, buf_out, sem):
        wid = lax.axis_index("s") * _NC + lax.axis_index("c")
        lanes = lax.iota(jnp.int32, 16)
        keep = lanes < EMBED_DIM
        zero = jnp.zeros((16,), jnp.float32)

        def step(i, carry):
            ch = wid + i * _NW

            @pl.when(ch < _PAD_NCH)
            def _():
                cps = [
                    pltpu.async_copy(
                        src_hbm.at[pl.ds(k * VOCAB + ch * _PAD_CH, _PAD_CH)],
                        buf_in.at[pl.ds(k * _PAD_CH, _PAD_CH)],
                        sem,
                    )
                    for k in range(EMBED_DIM)
                ]
                for cp in cps:
                    cp.wait()

                def rows(r, c):
                    for u in range(_PAD_UNROLL):
                        rr = r * _PAD_UNROLL + u
                        src_idx = jnp.minimum(rr + lanes * _PAD_CH, in_w - 1)
                        v = plsc.load_gather(buf_in, [src_idx])
                        v = jnp.where(keep, v, zero)
                        buf_out[pl.ds(rr * EMBED_PAD, 16)] = v
                    return c

                lax.fori_loop(0, _PAD_CH // _PAD_UNROLL, rows, 0)
                pltpu.sync_copy(buf_out, out_hbm.at[pl.ds(ch * out_w, out_w)])

            return carry

        lax.fori_loop(0, _PAD_ITERS, step, 0)

    return pad_kernel(table_flat_cm)


def _sc_gather(table16, idx2d):
    mesh = plsc.VectorSubcoreMesh(**_SC_MESH)

    @functools.partial(
        pl.kernel,
        mesh=mesh,
        out_type=jax.ShapeDtypeStruct((N_ROWS // _BLK, _BLK, EMBED_PAD), jnp.float32),
        scratch_types=[
            pltpu.VMEM((_FIRE, _BLK), jnp.int32),
            pltpu.VMEM((_FIRE, _BLK, EMBED_PAD), jnp.float32),
            pltpu.SemaphoreType.DMA,
        ],
        compiler_params=pltpu.CompilerParams(use_tc_tiling_on_sc=False),
    )
    def gather_kernel(table_hbm, idx_hbm, out_hbm, idx_v, rows_v, sem):
        wid = lax.axis_index("s") * _NC + lax.axis_index("c")
        base = wid * (_PER_W // _BLK)

        def step(i, carry):
            blk = base + i * _FIRE
            pltpu.sync_copy(idx_hbm.at[pl.ds(blk, _FIRE)], idx_v)
            cps = [
                pltpu.async_copy(table_hbm.at[idx_v.at[b]], rows_v.at[b], sem)
                for b in range(_FIRE)
            ]
            for cp in cps:
                cp.wait()
            pltpu.sync_copy(rows_v, out_hbm.at[pl.ds(blk, _FIRE)])
            return carry

        lax.fori_loop(0, _STEPS, step, 0)

    return gather_kernel(table16, idx2d)


_T = 2048


def _mlp_body(g_ref, w1_ref, b1_ref, w2_ref, b2_ref, w3_ref, b3_ref,
              w4_ref, b4_ref, o_ref):
    h = g_ref[...]
    h = jnp.maximum(
        jnp.dot(h, w1_ref[...], preferred_element_type=jnp.float32) + b1_ref[...], 0.0)
    h = jnp.maximum(
        jnp.dot(h, w2_ref[...], preferred_element_type=jnp.float32) + b2_ref[...], 0.0)
    h = jnp.tanh(
        jnp.dot(h, w3_ref[...], preferred_element_type=jnp.float32) + b3_ref[...])
    h = jnp.dot(h, w4_ref[...], preferred_element_type=jnp.float32) + b4_ref[...]
    o_ref[...] = jnp.maximum(h, 0.0)


def _blockdiag(W, p):
    k, m = W.shape
    eye = jnp.eye(p, dtype=W.dtype)
    return (eye[:, None, :, None] * W[None, :, None, :]).reshape(p * k, p * m)


def _tc_mlp(g2, W1big, b1big, W2big, b2big, W3big, b3big, W4big, b4big):
    full = lambda a: pl.BlockSpec(a.shape, lambda i: (0, 0))
    return pl.pallas_call(
        _mlp_body,
        grid=(N_PACKED // _T,),
        in_specs=[
            pl.BlockSpec((_T, PACK * EMBED_PAD), lambda i: (i, 0)),
            full(W1big), full(b1big),
            full(W2big), full(b2big),
            full(W3big), full(b3big),
            full(W4big), full(b4big),
        ],
        out_specs=pl.BlockSpec((_T, PACK), lambda i: (i, 0)),
        out_shape=jax.ShapeDtypeStruct((N_PACKED, PACK), jnp.float32),
        compiler_params=pltpu.CompilerParams(
            dimension_semantics=("parallel",),
        ),
    )(g2, W1big, b1big, W2big, b2big, W3big, b3big, W4big, b4big)


def kernel(x, table, W1, b1, W2, b2, W3, b3, W4, b4):
    idx2d = x.reshape(N_ROWS // _BLK, _BLK)
    table16 = _sc_pad(table.T.reshape(VOCAB * EMBED_DIM)).reshape(VOCAB, EMBED_PAD)
    g = _sc_gather(table16, idx2d)
    g2 = g.reshape(N_PACKED, PACK * EMBED_PAD)

    W1p = jnp.pad(W1, ((0, EMBED_PAD - EMBED_DIM), (0, 0)))
    W1big = _blockdiag(W1p, PACK)
    b1big = jnp.tile(b1, PACK).reshape(1, -1)
    W2big = _blockdiag(W2, PACK)
    b2big = jnp.tile(b2, PACK).reshape(1, -1)
    W3big = _blockdiag(W3, PACK)
    b3big = jnp.tile(b3, PACK).reshape(1, -1)
    W4big = _blockdiag(W4, PACK)
    b4big = jnp.tile(b4, PACK).reshape(1, -1)

    out = _tc_mlp(g2, W1big, b1big, W2big, b2big, W3big, b3big, W4big, b4big)
    return out.reshape(BATCH, FIELDS, 1)


# speedup vs baseline: 1.5719x; 1.5719x over previous
import functools

import jax
import jax.numpy as jnp
from jax import lax
from jax.experimental import pallas as pl
from jax.experimental.pallas import tpu as pltpu
from jax.experimental.pallas import tpu_sc as plsc

VOCAB = 1000000
EMBED_DIM = 15
EMBED_PAD = 16
BATCH = 16384
FIELDS = 100
N_ROWS = BATCH * FIELDS
PACK = 8
N_PACKED = N_ROWS // PACK

_NC = 2
_NS = 16
_NW = _NC * _NS
_FIRE = 8
_BLK = 128
_CHUNK = _FIRE * _BLK
_PER_W = N_ROWS // _NW
_STEPS = _PER_W // _CHUNK

_PAD_CH = 4000
_PAD_NCH = VOCAB // _PAD_CH
_PAD_ITERS = -(-_PAD_NCH // _NW)
_PAD_UNROLL = 8

_SC_MESH = dict(core_axis_name="c", subcore_axis_name="s")


def _sc_pad(table_flat):
    mesh = plsc.VectorSubcoreMesh(**_SC_MESH)
    in_w = _PAD_CH * EMBED_DIM
    out_w = _PAD_CH * EMBED_PAD

    @functools.partial(
        pl.kernel,
        mesh=mesh,
        out_type=jax.ShapeDtypeStruct((VOCAB * EMBED_PAD,), jnp.float32),
        scratch_types=[
            pltpu.VMEM((in_w,), jnp.float32),
            pltpu.VMEM((out_w,), jnp.float32),
        ],
        compiler_params=pltpu.CompilerParams(
            use_tc_tiling_on_sc=False, needs_layout_passes=False),
    )
    def pad_kernel(src_hbm, out_hbm, buf_in, buf_out):
        wid = lax.axis_index("s") * _NC + lax.axis_index("c")
        lanes = lax.iota(jnp.int32, 16)
        keep = lanes < EMBED_DIM
        zero = jnp.zeros((16,), jnp.float32)

        def step(i, carry):
            ch = wid + i * _NW

            @pl.when(ch < _PAD_NCH)
            def _():
                pltpu.sync_copy(src_hbm.at[pl.ds(ch * in_w, in_w)], buf_in)

                def rows(r, c):
                    for u in range(_PAD_UNROLL):
                        rw = (r * _PAD_UNROLL + u) * EMBED_DIM
                        src_idx = jnp.minimum(rw + lanes, in_w - 1)
                        v = plsc.load_gather(buf_in, [src_idx])
                        v = jnp.where(keep, v, zero)
                        buf_out[pl.ds((r * _PAD_UNROLL + u) * EMBED_PAD, 16)] = v
                    return c

                lax.fori_loop(0, _PAD_CH // _PAD_UNROLL, rows, 0)
                pltpu.sync_copy(buf_out, out_hbm.at[pl.ds(ch * out_w, out_w)])

            return carry

        lax.fori_loop(0, _PAD_ITERS, step, 0)

    return pad_kernel(table_flat)


def _sc_gather(table16, idx2d, nrows):
    mesh = plsc.VectorSubcoreMesh(**_SC_MESH)
    per_w_blk = nrows // _NW // _BLK
    steps = nrows // _NW // _CHUNK

    @functools.partial(
        pl.kernel,
        mesh=mesh,
        out_type=jax.ShapeDtypeStruct((nrows // _BLK, _BLK, EMBED_PAD), jnp.float32),
        scratch_types=[
            pltpu.VMEM((_FIRE, _BLK), jnp.int32),
            pltpu.VMEM((_FIRE, _BLK, EMBED_PAD), jnp.float32),
            pltpu.SemaphoreType.DMA,
        ],
        compiler_params=pltpu.CompilerParams(use_tc_tiling_on_sc=False),
    )
    def gather_kernel(table_hbm, idx_hbm, out_hbm, idx_v, rows_v, sem):
        wid = lax.axis_index("s") * _NC + lax.axis_index("c")
        base = wid * per_w_blk

        def step(i, carry):
            blk = base + i * _FIRE
            pltpu.sync_copy(idx_hbm.at[pl.ds(blk, _FIRE)], idx_v)
            cps = [
                pltpu.async_copy(table_hbm.at[idx_v.at[b]], rows_v.at[b], sem)
                for b in range(_FIRE)
            ]
            for cp in cps:
                cp.wait()
            pltpu.sync_copy(rows_v, out_hbm.at[pl.ds(blk, _FIRE)])
            return carry

        lax.fori_loop(0, steps, step, 0)

    return gather_kernel(table16, idx2d)


_T = 2048


def _mlp_body(g_ref, w1_ref, b1_ref, w2_ref, b2_ref, w3_ref, b3_ref,
              w4_ref, b4_ref, o_ref):
    h = g_ref[...]
    h = jnp.maximum(
        jnp.dot(h, w1_ref[...], preferred_element_type=jnp.float32) + b1_ref[...], 0.0)
    h = jnp.maximum(
        jnp.dot(h, w2_ref[...], preferred_element_type=jnp.float32) + b2_ref[...], 0.0)
    h = jnp.tanh(
        jnp.dot(h, w3_ref[...], preferred_element_type=jnp.float32) + b3_ref[...])
    h = jnp.dot(h, w4_ref[...], preferred_element_type=jnp.float32) + b4_ref[...]
    o_ref[...] = jnp.maximum(h, 0.0)


def _blockdiag(W, p):
    k, m = W.shape
    eye = jnp.eye(p, dtype=W.dtype)
    return (eye[:, None, :, None] * W[None, :, None, :]).reshape(p * k, p * m)


def _tc_mlp(g2, W1big, b1big, W2big, b2big, W3big, b3big, W4big, b4big):
    full = lambda a: pl.BlockSpec(a.shape, lambda i: (0, 0))
    npk = g2.shape[0]
    return pl.pallas_call(
        _mlp_body,
        grid=(npk // _T,),
        in_specs=[
            pl.BlockSpec((_T, PACK * EMBED_PAD), lambda i: (i, 0)),
            full(W1big), full(b1big),
            full(W2big), full(b2big),
            full(W3big), full(b3big),
            full(W4big), full(b4big),
        ],
        out_specs=pl.BlockSpec((_T, PACK), lambda i: (i, 0)),
        out_shape=jax.ShapeDtypeStruct((npk, PACK), jnp.float32),
        compiler_params=pltpu.CompilerParams(
            dimension_semantics=("parallel",),
        ),
    )(g2, W1big, b1big, W2big, b2big, W3big, b3big, W4big, b4big)


_SLICES = 4


def kernel(x, table, W1, b1, W2, b2, W3, b3, W4, b4):
    idx2d = x.reshape(N_ROWS // _BLK, _BLK)
    table16 = _sc_pad(table.reshape(VOCAB * EMBED_DIM)).reshape(VOCAB, EMBED_PAD)

    W1p = jnp.pad(W1, ((0, EMBED_PAD - EMBED_DIM), (0, 0)))
    W1big = _blockdiag(W1p, PACK)
    b1big = jnp.tile(b1, PACK).reshape(1, -1)
    W2big = _blockdiag(W2, PACK)
    b2big = jnp.tile(b2, PACK).reshape(1, -1)
    W3big = _blockdiag(W3, PACK)
    b3big = jnp.tile(b3, PACK).reshape(1, -1)
    W4big = _blockdiag(W4, PACK)
    b4big = jnp.tile(b4, PACK).reshape(1, -1)

    nsl = N_ROWS // _SLICES
    blk_sl = nsl // _BLK
    outs = []
    for s in range(_SLICES):
        g = _sc_gather(table16, idx2d[s * blk_sl:(s + 1) * blk_sl], nsl)
        g2 = g.reshape(nsl // PACK, PACK * EMBED_PAD)
        outs.append(
            _tc_mlp(g2, W1big, b1big, W2big, b2big, W3big, b3big, W4big, b4big))
    out = jnp.concatenate(outs, axis=0)
    return out.reshape(BATCH, FIELDS, 1)


# speedup vs baseline: 1.5883x; 1.0104x over previous
import functools

import jax
import jax.numpy as jnp
from jax import lax
from jax.experimental import pallas as pl
from jax.experimental.pallas import tpu as pltpu
from jax.experimental.pallas import tpu_sc as plsc

VOCAB = 1000000
EMBED_DIM = 15
EMBED_PAD = 16
BATCH = 16384
FIELDS = 100
N_ROWS = BATCH * FIELDS
PACK = 8
N_PACKED = N_ROWS // PACK

_NC = 2
_NS = 16
_NW = _NC * _NS
_FIRE = 8
_BLK = 128
_CHUNK = _FIRE * _BLK
_PER_W = N_ROWS // _NW
_STEPS = _PER_W // _CHUNK

_PAD_CH = 4000
_PAD_NCH = VOCAB // _PAD_CH
_PAD_ITERS = -(-_PAD_NCH // _NW)
_PAD_UNROLL = 8

_SC_MESH = dict(core_axis_name="c", subcore_axis_name="s")


def _sc_pad(table_flat):
    mesh = plsc.VectorSubcoreMesh(**_SC_MESH)
    in_w = _PAD_CH * EMBED_DIM
    out_w = _PAD_CH * EMBED_PAD

    @functools.partial(
        pl.kernel,
        mesh=mesh,
        out_type=jax.ShapeDtypeStruct((VOCAB * EMBED_PAD,), jnp.float32),
        scratch_types=[
            pltpu.VMEM((in_w,), jnp.float32),
            pltpu.VMEM((out_w,), jnp.float32),
        ],
        compiler_params=pltpu.CompilerParams(
            use_tc_tiling_on_sc=False, needs_layout_passes=False),
    )
    def pad_kernel(src_hbm, out_hbm, buf_in, buf_out):
        wid = lax.axis_index("s") * _NC + lax.axis_index("c")
        lanes = lax.iota(jnp.int32, 16)
        keep = lanes < EMBED_DIM
        zero = jnp.zeros((16,), jnp.float32)

        def step(i, carry):
            ch = wid + i * _NW

            @pl.when(ch < _PAD_NCH)
            def _():
                pltpu.sync_copy(src_hbm.at[pl.ds(ch * in_w, in_w)], buf_in)

                def rows(r, c):
                    for u in range(_PAD_UNROLL):
                        rw = (r * _PAD_UNROLL + u) * EMBED_DIM
                        src_idx = jnp.minimum(rw + lanes, in_w - 1)
                        v = plsc.load_gather(buf_in, [src_idx])
                        v = jnp.where(keep, v, zero)
                        buf_out[pl.ds((r * _PAD_UNROLL + u) * EMBED_PAD, 16)] = v
                    return c

                lax.fori_loop(0, _PAD_CH // _PAD_UNROLL, rows, 0)
                pltpu.sync_copy(buf_out, out_hbm.at[pl.ds(ch * out_w, out_w)])

            return carry

        lax.fori_loop(0, _PAD_ITERS, step, 0)

    return pad_kernel(table_flat)


def _sc_gather(table16, idx2d, nrows):
    mesh = plsc.VectorSubcoreMesh(**_SC_MESH)
    per_w_blk = nrows // _NW // _BLK
    steps = nrows // _NW // _CHUNK
    assert steps * _CHUNK * _NW == nrows, "slice must divide into full chunks"

    @functools.partial(
        pl.kernel,
        mesh=mesh,
        out_type=jax.ShapeDtypeStruct((nrows // _BLK, _BLK, EMBED_PAD), jnp.float32),
        scratch_types=[
            pltpu.VMEM((_FIRE, _BLK), jnp.int32),
            pltpu.VMEM((_FIRE, _BLK, EMBED_PAD), jnp.float32),
            pltpu.SemaphoreType.DMA,
        ],
        compiler_params=pltpu.CompilerParams(use_tc_tiling_on_sc=False),
    )
    def gather_kernel(table_hbm, idx_hbm, out_hbm, idx_v, rows_v, sem):
        wid = lax.axis_index("s") * _NC + lax.axis_index("c")
        base = wid * per_w_blk

        def step(i, carry):
            blk = base + i * _FIRE
            pltpu.sync_copy(idx_hbm.at[pl.ds(blk, _FIRE)], idx_v)
            cps = [
                pltpu.async_copy(table_hbm.at[idx_v.at[b]], rows_v.at[b], sem)
                for b in range(_FIRE)
            ]
            for cp in cps:
                cp.wait()
            pltpu.sync_copy(rows_v, out_hbm.at[pl.ds(blk, _FIRE)])
            return carry

        lax.fori_loop(0, steps, step, 0)

    return gather_kernel(table16, idx2d)


_T = 2048


def _mlp_body(g_ref, w1_ref, b1_ref, w2_ref, b2_ref, w3_ref, b3_ref,
              w4_ref, b4_ref, o_ref):
    h = g_ref[...]
    h = jnp.maximum(
        jnp.dot(h, w1_ref[...], preferred_element_type=jnp.float32) + b1_ref[...], 0.0)
    h = jnp.maximum(
        jnp.dot(h, w2_ref[...], preferred_element_type=jnp.float32) + b2_ref[...], 0.0)
    h = jnp.tanh(
        jnp.dot(h, w3_ref[...], preferred_element_type=jnp.float32) + b3_ref[...])
    h = jnp.dot(h, w4_ref[...], preferred_element_type=jnp.float32) + b4_ref[...]
    o_ref[...] = jnp.maximum(h, 0.0)


def _blockdiag(W, p):
    k, m = W.shape
    eye = jnp.eye(p, dtype=W.dtype)
    return (eye[:, None, :, None] * W[None, :, None, :]).reshape(p * k, p * m)


def _tc_mlp(g2, W1big, b1big, W2big, b2big, W3big, b3big, W4big, b4big):
    full = lambda a: pl.BlockSpec(a.shape, lambda i: (0, 0))
    npk = g2.shape[0]
    return pl.pallas_call(
        _mlp_body,
        grid=(npk // _T,),
        in_specs=[
            pl.BlockSpec((_T, PACK * EMBED_PAD), lambda i: (i, 0)),
            full(W1big), full(b1big),
            full(W2big), full(b2big),
            full(W3big), full(b3big),
            full(W4big), full(b4big),
        ],
        out_specs=pl.BlockSpec((_T, PACK), lambda i: (i, 0)),
        out_shape=jax.ShapeDtypeStruct((npk, PACK), jnp.float32),
        compiler_params=pltpu.CompilerParams(
            dimension_semantics=("parallel",),
        ),
    )(g2, W1big, b1big, W2big, b2big, W3big, b3big, W4big, b4big)


_SLICES = 5


def kernel(x, table, W1, b1, W2, b2, W3, b3, W4, b4):
    idx2d = x.reshape(N_ROWS // _BLK, _BLK)
    table16 = _sc_pad(table.reshape(VOCAB * EMBED_DIM)).reshape(VOCAB, EMBED_PAD)

    W1p = jnp.pad(W1, ((0, EMBED_PAD - EMBED_DIM), (0, 0)))
    W1big = _blockdiag(W1p, PACK)
    b1big = jnp.tile(b1, PACK).reshape(1, -1)
    W2big = _blockdiag(W2, PACK)
    b2big = jnp.tile(b2, PACK).reshape(1, -1)
    W3big = _blockdiag(W3, PACK)
    b3big = jnp.tile(b3, PACK).reshape(1, -1)
    W4big = _blockdiag(W4, PACK)
    b4big = jnp.tile(b4, PACK).reshape(1, -1)

    nsl = N_ROWS // _SLICES
    blk_sl = nsl // _BLK
    outs = []
    for s in range(_SLICES):
        g = _sc_gather(table16, idx2d[s * blk_sl:(s + 1) * blk_sl], nsl)
        g2 = g.reshape(nsl // PACK, PACK * EMBED_PAD)
        outs.append(
            _tc_mlp(g2, W1big, b1big, W2big, b2big, W3big, b3big, W4big, b4big))
    out = jnp.concatenate(outs, axis=0)
    return out.reshape(BATCH, FIELDS, 1)
